# baseline (speedup 1.0000x reference)
import functools

import jax
import jax.numpy as jnp
from jax import lax
from jax.experimental import pallas as pl
from jax.experimental.pallas import tpu as pltpu
from jax.experimental.pallas import tpu_sc as plsc

_DIM = 32
_N = 8192
_K = 8192

_TM = 512
_TK = 2048
_NT = _N // _TM
_KT = _K // _TK


def _argmin_body(x_ref, e_ref, ind_ref, diff_ref, best_ref, bidx_ref, acc_ref):
    i = pl.program_id(0)
    j = pl.program_id(1)
    xp = x_ref[...]
    ep = e_ref[...]
    xb = xp.astype(jnp.bfloat16).astype(jnp.float32)
    ab = lax.dot_general(xb, ep, (((1,), (0,)), ((), ())),
                         preferred_element_type=jnp.float32)
    x2 = jnp.sum(xp * xp, axis=1, keepdims=True)
    e2 = jnp.sum(ep * ep, axis=0, keepdims=True)
    d = (x2 - 2.0 * ab) + e2
    m = jnp.min(d, axis=1, keepdims=True)
    iota = lax.broadcasted_iota(jnp.int32, d.shape, 1)
    big = jnp.int32(2 ** 30)
    li = jnp.min(jnp.where(d == m, iota, big), axis=1, keepdims=True)
    li = li + j * _TK

    @pl.when(j == 0)
    def _():
        best_ref[...] = m
        bidx_ref[...] = li

    @pl.when(j > 0)
    def _():
        prev = best_ref[...].astype(jnp.bfloat16).astype(jnp.float32)
        better = m < prev
        bidx_ref[...] = jnp.where(better, li, bidx_ref[...])
        best_ref[...] = jnp.where(better, m, best_ref[...])

    @pl.when(j == _KT - 1)
    def _():
        ind_ref[...] = bidx_ref[...]
        part = jnp.sum(best_ref[...])

        @pl.when(i == 0)
        def _():
            acc_ref[0, 0] = part

        @pl.when(i > 0)
        def _():
            acc_ref[0, 0] = acc_ref[0, 0] + part

        @pl.when(i == _NT - 1)
        def _():
            diff_ref[0, 0] = acc_ref[0, 0]


def _argmin_call(flatten, embed):
    return pl.pallas_call(
        _argmin_body,
        grid=(_NT, _KT),
        in_specs=[
            pl.BlockSpec((_TM, 128), lambda i, j: (i, 0)),
            pl.BlockSpec((128, _TK), lambda i, j: (0, j)),
        ],
        out_specs=[
            pl.BlockSpec((_TM, 1), lambda i, j: (i, 0)),
            pl.BlockSpec(memory_space=pltpu.SMEM),
        ],
        out_shape=[
            jax.ShapeDtypeStruct((_N, 1), jnp.int32),
            jax.ShapeDtypeStruct((1, 1), jnp.float32),
        ],
        scratch_shapes=[
            pltpu.VMEM((_TM, 1), jnp.float32),
            pltpu.VMEM((_TM, 1), jnp.int32),
            pltpu.SMEM((1, 1), jnp.float32),
        ],
        compiler_params=pltpu.CompilerParams(
            dimension_semantics=("arbitrary", "arbitrary"),
        ),
    )(flatten, embed)


_NW = 32
_BPW = _N // _NW
_CH = 128
_NCH = _BPW // _CH


def _gather_call(table_p, idx3):
    mesh = plsc.VectorSubcoreMesh(core_axis_name="c", subcore_axis_name="s")

    @functools.partial(
        pl.kernel,
        mesh=mesh,
        out_type=jax.ShapeDtypeStruct((_N, 128), jnp.float32),
        scratch_types=[
            pltpu.VMEM((_NCH, _CH), jnp.int32),
            pltpu.VMEM((_CH, 128), jnp.float32),
            pltpu.SemaphoreType.DMA,
        ],
    )
    def gather_k(table_hbm, idx_hbm, out_hbm, idx_v, rows_v, sem):
        wid = lax.axis_index("s") * 2 + lax.axis_index("c")
        base = wid * _BPW
        pltpu.sync_copy(idx_hbm.at[wid], idx_v)
        for c in range(_NCH):
            pltpu.async_copy(table_hbm.at[idx_v.at[c]], rows_v, sem).wait()
            pltpu.sync_copy(rows_v, out_hbm.at[pl.ds(base + c * _CH, _CH)])

    return gather_k(table_p, idx3)


def kernel(input, embed):
    B, D, H, W = input.shape
    K = embed.shape[1]
    assert D == _DIM and K == _K and B * H * W == _N
    flatten = jnp.transpose(input, (0, 2, 3, 1)).reshape(_N, D)
    flatten_p = jnp.pad(flatten, ((0, 0), (0, 128 - _DIM)))
    embed_p = jnp.pad(embed, ((0, 128 - _DIM), (0, 0)))
    ind2d, diff2 = _argmin_call(flatten_p, embed_p)
    ind = ind2d.reshape(_N)
    table_p = jnp.pad(embed.T, ((0, 0), (0, 128 - _DIM)))
    q_rows = _gather_call(table_p, ind.reshape(_NW, _NCH, _CH))[:, :_DIM]
    quantize = q_rows.reshape(B, H, W, D).transpose(0, 3, 1, 2)
    diff = (diff2.reshape(()) / jnp.float32(_N * D)).astype(jnp.float32)
    return quantize, diff, ind.reshape(B, H, W)


# speedup vs baseline: 1.0598x; 1.0598x over previous
import functools

import jax
import jax.numpy as jnp
from jax import lax
from jax.experimental import pallas as pl
from jax.experimental.pallas import tpu as pltpu
from jax.experimental.pallas import tpu_sc as plsc

_DIM = 32
_N = 8192
_K = 8192

_TM = 1024
_TK = 2048
_NT = _N // _TM
_KT = _K // _TK


def _argmin_body(x_ref, e_ref, ind_ref, diff_ref, best_ref, bidx_ref, acc_ref):
    i = pl.program_id(0)
    j = pl.program_id(1)
    xp = x_ref[...]
    ep = e_ref[...]
    xb = xp.astype(jnp.bfloat16).astype(jnp.float32)
    ab2 = lax.dot_general(xb + xb, ep, (((1,), (0,)), ((), ())),
                          preferred_element_type=jnp.float32)
    x2 = jnp.sum(xp * xp, axis=1, keepdims=True)
    e2 = jnp.sum(ep * ep, axis=0, keepdims=True)
    d = (x2 - ab2) + e2
    m = jnp.min(d, axis=1, keepdims=True)
    iota = lax.broadcasted_iota(jnp.int32, d.shape, 1)
    big = jnp.int32(2 ** 30)
    li = jnp.min(jnp.where(d == m, iota, big), axis=1, keepdims=True)
    li = li + j * _TK

    @pl.when(j == 0)
    def _():
        best_ref[...] = m
        bidx_ref[...] = li

    @pl.when(j > 0)
    def _():
        prev = best_ref[...].astype(jnp.bfloat16).astype(jnp.float32)
        better = m < prev
        bidx_ref[...] = jnp.where(better, li, bidx_ref[...])
        best_ref[...] = jnp.where(better, m, best_ref[...])

    @pl.when(j == _KT - 1)
    def _():
        ind_ref[...] = bidx_ref[...]
        part = jnp.sum(best_ref[...])

        @pl.when(i == 0)
        def _():
            acc_ref[0, 0] = part

        @pl.when(i > 0)
        def _():
            acc_ref[0, 0] = acc_ref[0, 0] + part

        @pl.when(i == _NT - 1)
        def _():
            diff_ref[0, 0] = acc_ref[0, 0]


def _argmin_call(flatten, embed):
    return pl.pallas_call(
        _argmin_body,
        grid=(_NT, _KT),
        in_specs=[
            pl.BlockSpec((_TM, 128), lambda i, j: (i, 0)),
            pl.BlockSpec((128, _TK), lambda i, j: (0, j)),
        ],
        out_specs=[
            pl.BlockSpec((_TM, 1), lambda i, j: (i, 0)),
            pl.BlockSpec(memory_space=pltpu.SMEM),
        ],
        out_shape=[
            jax.ShapeDtypeStruct((_N, 1), jnp.int32),
            jax.ShapeDtypeStruct((1, 1), jnp.float32),
        ],
        scratch_shapes=[
            pltpu.VMEM((_TM, 1), jnp.float32),
            pltpu.VMEM((_TM, 1), jnp.int32),
            pltpu.SMEM((1, 1), jnp.float32),
        ],
        compiler_params=pltpu.CompilerParams(
            dimension_semantics=("arbitrary", "arbitrary"),
        ),
    )(flatten, embed)


_NW = 32
_BPW = _N // _NW
_CH = 128
_NCH = _BPW // _CH


def _gather_call(table_p, idx3):
    mesh = plsc.VectorSubcoreMesh(core_axis_name="c", subcore_axis_name="s")

    @functools.partial(
        pl.kernel,
        mesh=mesh,
        out_type=jax.ShapeDtypeStruct((_N, 128), jnp.float32),
        scratch_types=[
            pltpu.VMEM((_NCH, _CH), jnp.int32),
            pltpu.VMEM((_CH, 128), jnp.float32),
            pltpu.SemaphoreType.DMA,
        ],
    )
    def gather_k(table_hbm, idx_hbm, out_hbm, idx_v, rows_v, sem):
        wid = lax.axis_index("s") * 2 + lax.axis_index("c")
        base = wid * _BPW
        pltpu.sync_copy(idx_hbm.at[wid], idx_v)
        for c in range(_NCH):
            pltpu.async_copy(table_hbm.at[idx_v.at[c]], rows_v, sem).wait()
            pltpu.sync_copy(rows_v, out_hbm.at[pl.ds(base + c * _CH, _CH)])

    return gather_k(table_p, idx3)


def kernel(input, embed):
    B, D, H, W = input.shape
    K = embed.shape[1]
    assert D == _DIM and K == _K and B * H * W == _N
    flatten = jnp.transpose(input, (0, 2, 3, 1)).reshape(_N, D)
    flatten_p = jnp.pad(flatten, ((0, 0), (0, 128 - _DIM)))
    embed_p = jnp.pad(embed, ((0, 128 - _DIM), (0, 0)))
    ind2d, diff2 = _argmin_call(flatten_p, embed_p)
    ind = ind2d.reshape(_N)
    table_p = jnp.pad(embed.T, ((0, 0), (0, 128 - _DIM)))
    q_rows = _gather_call(table_p, ind.reshape(_NW, _NCH, _CH))[:, :_DIM]
    quantize = q_rows.reshape(B, H, W, D).transpose(0, 3, 1, 2)
    diff = (diff2.reshape(()) / jnp.float32(_N * D)).astype(jnp.float32)
    return quantize, diff, ind.reshape(B, H, W)


# speedup vs baseline: 1.1649x; 1.0991x over previous
import functools

import jax
import jax.numpy as jnp
from jax import lax
from jax.experimental import pallas as pl
from jax.experimental.pallas import tpu as pltpu
from jax.experimental.pallas import tpu_sc as plsc

_DIM = 32
_N = 8192
_K = 8192

_TM = 1024
_TK = 2048
_NT = _N // _TM
_KT = _K // _TK


def _argmin_body(x_ref, e_ref, ind_ref, diff_ref, best_ref, bidx_ref, acc_ref,
                 iota_ref):
    i = pl.program_id(0)
    j = pl.program_id(1)

    @pl.when(jnp.logical_and(i == 0, j == 0))
    def _():
        iota_ref[...] = lax.broadcasted_iota(
            jnp.int32, (1, _TK), 1).astype(jnp.float32)
    xp = x_ref[...]
    ep = e_ref[...]
    xb = xp.astype(jnp.bfloat16).astype(jnp.float32)
    ab2 = lax.dot_general(xb + xb, ep, (((1,), (0,)), ((), ())),
                          preferred_element_type=jnp.float32)
    x2 = jnp.sum(xp * xp, axis=1, keepdims=True)
    e2 = jnp.sum(ep * ep, axis=0, keepdims=True)
    d = (x2 - ab2) + e2
    m = jnp.min(d, axis=1, keepdims=True)
    iota = jnp.broadcast_to(iota_ref[...], d.shape)
    lif = jnp.min(jnp.where(d == m, iota, jnp.float32(3e38)),
                  axis=1, keepdims=True)
    li = lif.astype(jnp.int32) + j * _TK

    @pl.when(j == 0)
    def _():
        best_ref[...] = m
        bidx_ref[...] = li

    @pl.when(j > 0)
    def _():
        prev = best_ref[...].astype(jnp.bfloat16).astype(jnp.float32)
        better = m < prev
        bidx_ref[...] = jnp.where(better, li, bidx_ref[...])
        best_ref[...] = jnp.where(better, m, best_ref[...])

    @pl.when(j == _KT - 1)
    def _():
        ind_ref[...] = bidx_ref[...]
        part = jnp.sum(best_ref[...])

        @pl.when(i == 0)
        def _():
            acc_ref[0, 0] = part

        @pl.when(i > 0)
        def _():
            acc_ref[0, 0] = acc_ref[0, 0] + part

        @pl.when(i == _NT - 1)
        def _():
            diff_ref[0, 0] = acc_ref[0, 0]


def _argmin_call(flatten, embed):
    return pl.pallas_call(
        _argmin_body,
        grid=(_NT, _KT),
        in_specs=[
            pl.BlockSpec((_TM, 128), lambda i, j: (i, 0)),
            pl.BlockSpec((128, _TK), lambda i, j: (0, j)),
        ],
        out_specs=[
            pl.BlockSpec((_TM, 1), lambda i, j: (i, 0)),
            pl.BlockSpec(memory_space=pltpu.SMEM),
        ],
        out_shape=[
            jax.ShapeDtypeStruct((_N, 1), jnp.int32),
            jax.ShapeDtypeStruct((1, 1), jnp.float32),
        ],
        scratch_shapes=[
            pltpu.VMEM((_TM, 1), jnp.float32),
            pltpu.VMEM((_TM, 1), jnp.int32),
            pltpu.SMEM((1, 1), jnp.float32),
            pltpu.VMEM((1, _TK), jnp.float32),
        ],
        compiler_params=pltpu.CompilerParams(
            dimension_semantics=("arbitrary", "arbitrary"),
        ),
    )(flatten, embed)


_NW = 32
_BPW = _N // _NW
_CH = 128
_NCH = _BPW // _CH


def _gather_call(table_p, idx3):
    mesh = plsc.VectorSubcoreMesh(core_axis_name="c", subcore_axis_name="s")

    @functools.partial(
        pl.kernel,
        mesh=mesh,
        out_type=jax.ShapeDtypeStruct((_N, 128), jnp.float32),
        scratch_types=[
            pltpu.VMEM((_NCH, _CH), jnp.int32),
            pltpu.VMEM((_CH, 128), jnp.float32),
            pltpu.SemaphoreType.DMA,
        ],
    )
    def gather_k(table_hbm, idx_hbm, out_hbm, idx_v, rows_v, sem):
        wid = lax.axis_index("s") * 2 + lax.axis_index("c")
        base = wid * _BPW
        pltpu.sync_copy(idx_hbm.at[wid], idx_v)
        for c in range(_NCH):
            pltpu.async_copy(table_hbm.at[idx_v.at[c]], rows_v, sem).wait()
            pltpu.sync_copy(rows_v, out_hbm.at[pl.ds(base + c * _CH, _CH)])

    return gather_k(table_p, idx3)


def kernel(input, embed):
    B, D, H, W = input.shape
    K = embed.shape[1]
    assert D == _DIM and K == _K and B * H * W == _N
    flatten = jnp.transpose(input, (0, 2, 3, 1)).reshape(_N, D)
    flatten_p = jnp.pad(flatten, ((0, 0), (0, 128 - _DIM)))
    embed_p = jnp.pad(embed, ((0, 128 - _DIM), (0, 0)))
    ind2d, diff2 = _argmin_call(flatten_p, embed_p)
    ind = ind2d.reshape(_N)
    table_p = jnp.pad(embed.T, ((0, 0), (0, 128 - _DIM)))
    q_rows = _gather_call(table_p, ind.reshape(_NW, _NCH, _CH))[:, :_DIM]
    quantize = q_rows.reshape(B, H, W, D).transpose(0, 3, 1, 2)
    diff = (diff2.reshape(()) / jnp.float32(_N * D)).astype(jnp.float32)
    return quantize, diff, ind.reshape(B, H, W)
